# baseline (device time: 99810 ns/iter reference)
import jax
import jax.numpy as jnp
from jax import lax
from jax.experimental import pallas as pl
from jax.experimental.pallas import tpu as pltpu

N_DEV = 4


def kernel(x, w_mat):
    m_global, _ = x.shape
    _, n = w_mat.shape
    m_per = m_global // N_DEV
    h = n // 2
    hh = h // 2

    rings = [
        dict(idx=0, dir_right=True, q=0),
        dict(idx=1, dir_right=False, q=0),
        dict(idx=2, dir_right=True, q=1),
        dict(idx=3, dir_right=False, q=1),
    ]

    def body(
        x_ref,
        w_ref,
        out_ref,
        send0_ref,
        recv_ref,
        amax_ref,
        send_sems,
        recv_sems,
        amax_send_sems,
        amax_recv_sems,
    ):
        my = lax.axis_index("i")
        right = lax.rem(my + 1, N_DEV)
        left = lax.rem(my - 1 + N_DEV, N_DEV)

        barrier_sem = pltpu.get_barrier_semaphore()
        for nbr in (left, right):
            pl.semaphore_signal(
                barrier_sem,
                inc=1,
                device_id=(nbr,),
                device_id_type=pl.DeviceIdType.MESH,
            )
        pl.semaphore_wait(barrier_sem, 2)

        w_bf = w_ref[...].astype(jnp.bfloat16)

        def half_partial(j, dir_right):
            xc = x_ref[pl.ds(j * m_per, m_per), :].astype(jnp.bfloat16)
            lo = 0 if dir_right else h
            return jnp.dot(
                xc, w_bf[:, lo : lo + h], preferred_element_type=jnp.float32
            )

        rdmas = {r["idx"]: [] for r in rings}
        for s in range(N_DEV - 1):
            jr = lax.rem(my - (s + 1) + N_DEV, N_DEV)
            jl = lax.rem(my + s + 1, N_DEV)
            part_r = half_partial(jr, True)
            part_l = half_partial(jl, False)
            for ring in rings:
                r = ring["idx"]
                q0 = ring["q"] * hh
                part = (part_r if ring["dir_right"] else part_l)[:, q0 : q0 + hh]
                if s == 0:
                    send0_ref[r] = part.astype(jnp.bfloat16)
                    src = send0_ref.at[r]
                else:
                    rdmas[r][s - 1].wait_recv()
                    acc = recv_ref[r, s - 1].astype(jnp.float32) + part
                    recv_ref[r, s - 1] = acc.astype(jnp.bfloat16)
                    src = recv_ref.at[r, s - 1]
                rdma = pltpu.make_async_remote_copy(
                    src_ref=src,
                    dst_ref=recv_ref.at[r, s],
                    send_sem=send_sems.at[r, s],
                    recv_sem=recv_sems.at[r, s],
                    device_id=(right if ring["dir_right"] else left,),
                    device_id_type=pl.DeviceIdType.MESH,
                )
                rdma.start()
                rdmas[r].append(rdma)

        part_r = half_partial(my, True)
        part_l = half_partial(my, False)
        amax = jnp.float32(0.0)
        for ring in rings:
            r = ring["idx"]
            q0 = ring["q"] * hh
            part = (part_r if ring["dir_right"] else part_l)[:, q0 : q0 + hh]
            rdmas[r][-1].wait_recv()
            y = jnp.maximum(recv_ref[r, N_DEV - 2].astype(jnp.float32) + part, 0.0)
            lo = (0 if ring["dir_right"] else h) + q0
            out_ref[:, lo : lo + hh] = y
            amax = jnp.maximum(amax, jnp.max(y))

        amax_ref[0] = jnp.full((8, 128), amax, jnp.float32)
        amax_rdmas = []
        for d in range(1, N_DEV):
            peer = lax.rem(my + d, N_DEV)
            rr = pltpu.make_async_remote_copy(
                src_ref=amax_ref.at[0],
                dst_ref=amax_ref.at[d],
                send_sem=amax_send_sems.at[d - 1],
                recv_sem=amax_recv_sems.at[d - 1],
                device_id=(peer,),
                device_id_type=pl.DeviceIdType.MESH,
            )
            rr.start()
            amax_rdmas.append(rr)
        for rr in amax_rdmas:
            rr.wait_recv()

        gmax = jnp.max(amax_ref[...])
        scale = gmax / 127.0
        inv = 127.0 / gmax
        q = jnp.round(jnp.clip(out_ref[...] * inv, 0.0, 127.0))
        out_ref[...] = q * scale

        for r in rdmas:
            for rd in rdmas[r]:
                rd.wait_send()
        for rr in amax_rdmas:
            rr.wait_send()

    return pl.pallas_call(
        body,
        out_shape=jax.ShapeDtypeStruct((m_per, n), jnp.float32),
        in_specs=[
            pl.BlockSpec(memory_space=pltpu.VMEM),
            pl.BlockSpec(memory_space=pltpu.VMEM),
        ],
        out_specs=pl.BlockSpec(memory_space=pltpu.VMEM),
        scratch_shapes=[
            pltpu.VMEM((4, m_per, hh), jnp.bfloat16),
            pltpu.VMEM((4, N_DEV - 1, m_per, hh), jnp.bfloat16),
            pltpu.VMEM((N_DEV, 8, 128), jnp.float32),
            pltpu.SemaphoreType.DMA((4, N_DEV - 1)),
            pltpu.SemaphoreType.DMA((4, N_DEV - 1)),
            pltpu.SemaphoreType.DMA((N_DEV - 1,)),
            pltpu.SemaphoreType.DMA((N_DEV - 1,)),
        ],
        compiler_params=pltpu.CompilerParams(
            vmem_limit_bytes=128 * 1024 * 1024,
            collective_id=0,
        ),
    )(x, w_mat)


# device time: 99791 ns/iter; 1.0002x vs baseline; 1.0002x over previous
import jax
import jax.numpy as jnp
from jax import lax
from jax.experimental import pallas as pl
from jax.experimental.pallas import tpu as pltpu

N_DEV = 4


def kernel(x, w_mat):
    m_global, _ = x.shape
    _, n = w_mat.shape
    m_per = m_global // N_DEV
    h = n // 2
    hh = h // 2

    rings = [
        dict(idx=0, dir_right=True, q=0),
        dict(idx=1, dir_right=False, q=0),
        dict(idx=2, dir_right=True, q=1),
        dict(idx=3, dir_right=False, q=1),
    ]

    def body(
        x_ref,
        w_ref,
        out_ref,
        send0_ref,
        recv_ref,
        amax_ref,
        send_sems,
        recv_sems,
        amax_send_sems,
        amax_recv_sems,
    ):
        my = lax.axis_index("i")
        right = lax.rem(my + 1, N_DEV)
        left = lax.rem(my - 1 + N_DEV, N_DEV)

        barrier_sem = pltpu.get_barrier_semaphore()
        for nbr in (left, right):
            pl.semaphore_signal(
                barrier_sem,
                inc=1,
                device_id=(nbr,),
                device_id_type=pl.DeviceIdType.MESH,
            )
        pl.semaphore_wait(barrier_sem, 2)

        w_bf = w_ref[...].astype(jnp.bfloat16)

        def half_partial(j, dir_right):
            xc = x_ref[pl.ds(j * m_per, m_per), :].astype(jnp.bfloat16)
            lo = 0 if dir_right else h
            return jnp.dot(
                xc, w_bf[:, lo : lo + h], preferred_element_type=jnp.float32
            ).astype(jnp.bfloat16)

        rdmas = {r["idx"]: [] for r in rings}
        for s in range(N_DEV - 1):
            jr = lax.rem(my - (s + 1) + N_DEV, N_DEV)
            jl = lax.rem(my + s + 1, N_DEV)
            part_r = half_partial(jr, True)
            part_l = half_partial(jl, False)
            for ring in rings:
                r = ring["idx"]
                q0 = ring["q"] * hh
                part = (part_r if ring["dir_right"] else part_l)[:, q0 : q0 + hh]
                if s == 0:
                    send0_ref[r] = part
                    src = send0_ref.at[r]
                else:
                    rdmas[r][s - 1].wait_recv()
                    recv_ref[r, s - 1] = recv_ref[r, s - 1] + part
                    src = recv_ref.at[r, s - 1]
                rdma = pltpu.make_async_remote_copy(
                    src_ref=src,
                    dst_ref=recv_ref.at[r, s],
                    send_sem=send_sems.at[r, s],
                    recv_sem=recv_sems.at[r, s],
                    device_id=(right if ring["dir_right"] else left,),
                    device_id_type=pl.DeviceIdType.MESH,
                )
                rdma.start()
                rdmas[r].append(rdma)

        part_r = half_partial(my, True)
        part_l = half_partial(my, False)
        amax = jnp.float32(0.0)
        for ring in rings:
            r = ring["idx"]
            q0 = ring["q"] * hh
            part = (part_r if ring["dir_right"] else part_l)[:, q0 : q0 + hh]
            rdmas[r][-1].wait_recv()
            yb = recv_ref[r, N_DEV - 2] + part
            y = jnp.maximum(yb, jnp.zeros_like(yb))
            lo = (0 if ring["dir_right"] else h) + q0
            out_ref[:, lo : lo + hh] = y.astype(jnp.float32)
            amax = jnp.maximum(amax, jnp.max(y.astype(jnp.float32)))

        amax_ref[0] = jnp.full((8, 128), amax, jnp.float32)
        amax_rdmas = []
        for d in range(1, N_DEV):
            peer = lax.rem(my + d, N_DEV)
            rr = pltpu.make_async_remote_copy(
                src_ref=amax_ref.at[0],
                dst_ref=amax_ref.at[d],
                send_sem=amax_send_sems.at[d - 1],
                recv_sem=amax_recv_sems.at[d - 1],
                device_id=(peer,),
                device_id_type=pl.DeviceIdType.MESH,
            )
            rr.start()
            amax_rdmas.append(rr)
        for rr in amax_rdmas:
            rr.wait_recv()

        gmax = jnp.max(amax_ref[...])
        scale = gmax / 127.0
        inv = 127.0 / gmax
        q = jnp.round(jnp.clip(out_ref[...] * inv, 0.0, 127.0))
        out_ref[...] = q * scale

        for r in rdmas:
            for rd in rdmas[r]:
                rd.wait_send()
        for rr in amax_rdmas:
            rr.wait_send()

    return pl.pallas_call(
        body,
        out_shape=jax.ShapeDtypeStruct((m_per, n), jnp.float32),
        in_specs=[
            pl.BlockSpec(memory_space=pltpu.VMEM),
            pl.BlockSpec(memory_space=pltpu.VMEM),
        ],
        out_specs=pl.BlockSpec(memory_space=pltpu.VMEM),
        scratch_shapes=[
            pltpu.VMEM((4, m_per, hh), jnp.bfloat16),
            pltpu.VMEM((4, N_DEV - 1, m_per, hh), jnp.bfloat16),
            pltpu.VMEM((N_DEV, 8, 128), jnp.float32),
            pltpu.SemaphoreType.DMA((4, N_DEV - 1)),
            pltpu.SemaphoreType.DMA((4, N_DEV - 1)),
            pltpu.SemaphoreType.DMA((N_DEV - 1,)),
            pltpu.SemaphoreType.DMA((N_DEV - 1,)),
        ],
        compiler_params=pltpu.CompilerParams(
            vmem_limit_bytes=128 * 1024 * 1024,
            collective_id=0,
        ),
    )(x, w_mat)


# device time: 94108 ns/iter; 1.0606x vs baseline; 1.0604x over previous
import jax
import jax.numpy as jnp
from jax import lax
from jax.experimental import pallas as pl
from jax.experimental.pallas import tpu as pltpu

N_DEV = 4


def kernel(x, w_mat):
    m_global, k = x.shape
    _, n = w_mat.shape
    m_per = m_global // N_DEV
    h = n // 2
    hh = h // 2

    rings = [
        dict(idx=0, dir_right=True, q=0),
        dict(idx=1, dir_right=False, q=0),
        dict(idx=2, dir_right=True, q=1),
        dict(idx=3, dir_right=False, q=1),
    ]

    def body(
        x_hbm,
        w_hbm,
        out_ref,
        xs_ref,
        w_ref,
        send0_ref,
        recv_ref,
        amax_ref,
        load_sems,
        send_sems,
        recv_sems,
        amax_send_sems,
        amax_recv_sems,
    ):
        my = lax.axis_index("i")
        right = lax.rem(my + 1, N_DEV)
        left = lax.rem(my - 1 + N_DEV, N_DEV)

        w_dma = pltpu.make_async_copy(w_hbm, w_ref, load_sems.at[0])
        w_dma.start()
        chunk_order = [1, 3, 2, 0]
        chunk_dmas = {}
        for i, d in enumerate(chunk_order):
            j = lax.rem(my - d + N_DEV, N_DEV)
            dma = pltpu.make_async_copy(
                x_hbm.at[pl.ds(j * m_per, m_per), :],
                xs_ref.at[pl.ds(j * m_per, m_per), :],
                load_sems.at[1 + i],
            )
            dma.start()
            chunk_dmas[d] = dma

        barrier_sem = pltpu.get_barrier_semaphore()
        for nbr in (left, right):
            pl.semaphore_signal(
                barrier_sem,
                inc=1,
                device_id=(nbr,),
                device_id_type=pl.DeviceIdType.MESH,
            )
        pl.semaphore_wait(barrier_sem, 2)

        w_dma.wait()
        w_halves = {}

        def half_partial(j, d, dir_right):
            if d in chunk_dmas:
                chunk_dmas[d].wait()
                del chunk_dmas[d]
            if dir_right not in w_halves:
                lo = 0 if dir_right else h
                w_halves[dir_right] = w_ref[:, lo : lo + h].astype(jnp.bfloat16)
            xc = xs_ref[pl.ds(j * m_per, m_per), :].astype(jnp.bfloat16)
            return jnp.dot(
                xc, w_halves[dir_right], preferred_element_type=jnp.float32
            ).astype(jnp.bfloat16)

        rdmas = {r["idx"]: [] for r in rings}
        for s in range(N_DEV - 1):
            jr = lax.rem(my - (s + 1) + N_DEV, N_DEV)
            jl = lax.rem(my + s + 1, N_DEV)
            part_r = half_partial(jr, s + 1, True)
            part_l = half_partial(jl, 3 - s, False)
            for ring in rings:
                r = ring["idx"]
                q0 = ring["q"] * hh
                part = (part_r if ring["dir_right"] else part_l)[:, q0 : q0 + hh]
                if s == 0:
                    send0_ref[r] = part
                    src = send0_ref.at[r]
                else:
                    rdmas[r][s - 1].wait_recv()
                    recv_ref[r, s - 1] = recv_ref[r, s - 1] + part
                    src = recv_ref.at[r, s - 1]
                rdma = pltpu.make_async_remote_copy(
                    src_ref=src,
                    dst_ref=recv_ref.at[r, s],
                    send_sem=send_sems.at[r, s],
                    recv_sem=recv_sems.at[r, s],
                    device_id=(right if ring["dir_right"] else left,),
                    device_id_type=pl.DeviceIdType.MESH,
                )
                rdma.start()
                rdmas[r].append(rdma)

        part_r = half_partial(my, 0, True)
        part_l = half_partial(my, 0, False)
        amax = jnp.float32(0.0)
        for ring in rings:
            r = ring["idx"]
            q0 = ring["q"] * hh
            part = (part_r if ring["dir_right"] else part_l)[:, q0 : q0 + hh]
            rdmas[r][-1].wait_recv()
            yb = recv_ref[r, N_DEV - 2] + part
            y = jnp.maximum(yb, jnp.zeros_like(yb))
            lo = (0 if ring["dir_right"] else h) + q0
            out_ref[:, lo : lo + hh] = y
            amax = jnp.maximum(amax, jnp.max(y.astype(jnp.float32)))

        amax_ref[0] = jnp.full((8, 128), amax, jnp.float32)
        amax_rdmas = []
        for d in range(1, N_DEV):
            peer = lax.rem(my + d, N_DEV)
            rr = pltpu.make_async_remote_copy(
                src_ref=amax_ref.at[0],
                dst_ref=amax_ref.at[d],
                send_sem=amax_send_sems.at[d - 1],
                recv_sem=amax_recv_sems.at[d - 1],
                device_id=(peer,),
                device_id_type=pl.DeviceIdType.MESH,
            )
            rr.start()
            amax_rdmas.append(rr)
        for rr in amax_rdmas:
            rr.wait_recv()

        gmax = jnp.max(amax_ref[...])
        scale = gmax / 127.0
        inv = 127.0 / gmax
        v = out_ref[...].astype(jnp.float32)
        q = jnp.round(jnp.clip(v * inv, 0.0, 127.0))
        out_ref[...] = (q * scale).astype(jnp.bfloat16)

        for r in rdmas:
            for rd in rdmas[r]:
                rd.wait_send()
        for rr in amax_rdmas:
            rr.wait_send()

    return pl.pallas_call(
        body,
        out_shape=jax.ShapeDtypeStruct((m_per, n), jnp.bfloat16),
        in_specs=[
            pl.BlockSpec(memory_space=pltpu.MemorySpace.HBM),
            pl.BlockSpec(memory_space=pltpu.MemorySpace.HBM),
        ],
        out_specs=pl.BlockSpec(memory_space=pltpu.VMEM),
        scratch_shapes=[
            pltpu.VMEM((m_global, k), jnp.float32),
            pltpu.VMEM((k, n), jnp.float32),
            pltpu.VMEM((4, m_per, hh), jnp.bfloat16),
            pltpu.VMEM((4, N_DEV - 1, m_per, hh), jnp.bfloat16),
            pltpu.VMEM((N_DEV, 8, 128), jnp.float32),
            pltpu.SemaphoreType.DMA((5,)),
            pltpu.SemaphoreType.DMA((4, N_DEV - 1)),
            pltpu.SemaphoreType.DMA((4, N_DEV - 1)),
            pltpu.SemaphoreType.DMA((N_DEV - 1,)),
            pltpu.SemaphoreType.DMA((N_DEV - 1,)),
        ],
        compiler_params=pltpu.CompilerParams(
            vmem_limit_bytes=128 * 1024 * 1024,
            collective_id=0,
        ),
    )(x, w_mat)


# device time: 92340 ns/iter; 1.0809x vs baseline; 1.0191x over previous
import jax
import jax.numpy as jnp
from jax import lax
from jax.experimental import pallas as pl
from jax.experimental.pallas import tpu as pltpu

N_DEV = 4


def kernel(x, w_mat):
    m_global, k = x.shape
    _, n = w_mat.shape
    m_per = m_global // N_DEV
    h = n // 2
    hh = h // 2

    rings = [
        dict(idx=0, dir_right=True, wq=0),
        dict(idx=1, dir_right=False, wq=2),
        dict(idx=2, dir_right=True, wq=1),
        dict(idx=3, dir_right=False, wq=3),
    ]
    for ring in rings:
        ring["lo"] = ring["wq"] * hh

    def body(
        x_hbm,
        w_hbm,
        out_ref,
        xs_ref,
        w_ref,
        send0_ref,
        recv_ref,
        amax_ref,
        load_sems,
        send_sems,
        recv_sems,
        amax_send_sems,
        amax_recv_sems,
    ):
        my = lax.axis_index("i")
        right = lax.rem(my + 1, N_DEV)
        left = lax.rem(my - 1 + N_DEV, N_DEV)

        w_dmas = {}
        for i, qi in enumerate([0, 2, 1, 3]):
            dma = pltpu.make_async_copy(
                w_hbm.at[:, pl.ds(qi * hh, hh)],
                w_ref.at[:, pl.ds(qi * hh, hh)],
                load_sems.at[i],
            )
            dma.start()
            w_dmas[qi] = dma
        chunk_order = [1, 3, 2, 0]
        chunk_dmas = {}
        for i, d in enumerate(chunk_order):
            j = lax.rem(my - d + N_DEV, N_DEV)
            dma = pltpu.make_async_copy(
                x_hbm.at[pl.ds(j * m_per, m_per), :],
                xs_ref.at[pl.ds(j * m_per, m_per), :],
                load_sems.at[4 + i],
            )
            dma.start()
            chunk_dmas[d] = dma

        barrier_sem = pltpu.get_barrier_semaphore()
        for nbr in (left, right):
            pl.semaphore_signal(
                barrier_sem,
                inc=1,
                device_id=(nbr,),
                device_id_type=pl.DeviceIdType.MESH,
            )
        pl.semaphore_wait(barrier_sem, 2)

        w_q = {}
        x_bf = {}

        def w_quarter(qi):
            if qi in w_dmas:
                w_dmas[qi].wait()
                del w_dmas[qi]
            if qi not in w_q:
                w_q[qi] = w_ref[:, qi * hh : (qi + 1) * hh].astype(jnp.bfloat16)
            return w_q[qi]

        def x_chunk(j, d):
            if d in chunk_dmas:
                chunk_dmas[d].wait()
                del chunk_dmas[d]
            if d not in x_bf:
                x_bf[d] = xs_ref[pl.ds(j * m_per, m_per), :].astype(jnp.bfloat16)
            return x_bf[d]

        def quarter_partial(ring, j, d):
            return jnp.dot(
                x_chunk(j, d),
                w_quarter(ring["wq"]),
                preferred_element_type=jnp.float32,
            ).astype(jnp.bfloat16)

        rdmas = {r["idx"]: [] for r in rings}
        for s in range(N_DEV - 1):
            jr = lax.rem(my - (s + 1) + N_DEV, N_DEV)
            jl = lax.rem(my + s + 1, N_DEV)
            for ring in rings:
                r = ring["idx"]
                j, d = (jr, s + 1) if ring["dir_right"] else (jl, 3 - s)
                part = quarter_partial(ring, j, d)
                if s == 0:
                    send0_ref[r] = part
                    src = send0_ref.at[r]
                else:
                    rdmas[r][s - 1].wait_recv()
                    recv_ref[r, s - 1] = recv_ref[r, s - 1] + part
                    src = recv_ref.at[r, s - 1]
                rdma = pltpu.make_async_remote_copy(
                    src_ref=src,
                    dst_ref=recv_ref.at[r, s],
                    send_sem=send_sems.at[r, s],
                    recv_sem=recv_sems.at[r, s],
                    device_id=(right if ring["dir_right"] else left,),
                    device_id_type=pl.DeviceIdType.MESH,
                )
                rdma.start()
                rdmas[r].append(rdma)

        amax = jnp.float32(0.0)
        for ring in rings:
            r = ring["idx"]
            part = quarter_partial(ring, my, 0)
            rdmas[r][-1].wait_recv()
            yb = recv_ref[r, N_DEV - 2] + part
            y = jnp.maximum(yb, jnp.zeros_like(yb))
            lo = ring["lo"]
            out_ref[:, lo : lo + hh] = y
            amax = jnp.maximum(amax, jnp.max(y.astype(jnp.float32)))

        amax_ref[0] = jnp.full((8, 128), amax, jnp.float32)
        amax_rdmas = []
        for d in range(1, N_DEV):
            peer = lax.rem(my + d, N_DEV)
            rr = pltpu.make_async_remote_copy(
                src_ref=amax_ref.at[0],
                dst_ref=amax_ref.at[d],
                send_sem=amax_send_sems.at[d - 1],
                recv_sem=amax_recv_sems.at[d - 1],
                device_id=(peer,),
                device_id_type=pl.DeviceIdType.MESH,
            )
            rr.start()
            amax_rdmas.append(rr)
        for rr in amax_rdmas:
            rr.wait_recv()

        gmax = jnp.max(amax_ref[...])
        scale = gmax / 127.0
        inv = 127.0 / gmax
        v = out_ref[...].astype(jnp.float32)
        q = jnp.round(jnp.clip(v * inv, 0.0, 127.0))
        out_ref[...] = (q * scale).astype(jnp.bfloat16)

        for r in rdmas:
            for rd in rdmas[r]:
                rd.wait_send()
        for rr in amax_rdmas:
            rr.wait_send()

    return pl.pallas_call(
        body,
        out_shape=jax.ShapeDtypeStruct((m_per, n), jnp.bfloat16),
        in_specs=[
            pl.BlockSpec(memory_space=pltpu.MemorySpace.HBM),
            pl.BlockSpec(memory_space=pltpu.MemorySpace.HBM),
        ],
        out_specs=pl.BlockSpec(memory_space=pltpu.VMEM),
        scratch_shapes=[
            pltpu.VMEM((m_global, k), jnp.float32),
            pltpu.VMEM((k, n), jnp.float32),
            pltpu.VMEM((4, m_per, hh), jnp.bfloat16),
            pltpu.VMEM((4, N_DEV - 1, m_per, hh), jnp.bfloat16),
            pltpu.VMEM((N_DEV, 8, 128), jnp.float32),
            pltpu.SemaphoreType.DMA((8,)),
            pltpu.SemaphoreType.DMA((4, N_DEV - 1)),
            pltpu.SemaphoreType.DMA((4, N_DEV - 1)),
            pltpu.SemaphoreType.DMA((N_DEV - 1,)),
            pltpu.SemaphoreType.DMA((N_DEV - 1,)),
        ],
        compiler_params=pltpu.CompilerParams(
            vmem_limit_bytes=128 * 1024 * 1024,
            collective_id=0,
        ),
    )(x, w_mat)


# device time: 92205 ns/iter; 1.0825x vs baseline; 1.0015x over previous
import jax
import jax.numpy as jnp
from jax import lax
from jax.experimental import pallas as pl
from jax.experimental.pallas import tpu as pltpu

N_DEV = 4


def kernel(x, w_mat):
    m_global, k = x.shape
    _, n = w_mat.shape
    m_per = m_global // N_DEV
    h = n // 2
    hh = h // 2

    rings = [
        dict(idx=0, dir_right=True, wq=0),
        dict(idx=1, dir_right=False, wq=2),
        dict(idx=2, dir_right=True, wq=1),
        dict(idx=3, dir_right=False, wq=3),
    ]
    for ring in rings:
        ring["lo"] = ring["wq"] * hh

    def body(
        x_hbm,
        w_hbm,
        out_ref,
        xs_ref,
        w_ref,
        send0_ref,
        recv_ref,
        amax_ref,
        load_sems,
        send_sems,
        recv_sems,
        amax_send_sems,
        amax_recv_sems,
    ):
        my = lax.axis_index("i")
        right = lax.rem(my + 1, N_DEV)
        left = lax.rem(my - 1 + N_DEV, N_DEV)

        w_dmas = {}
        for i, qi in enumerate([0, 2, 1, 3]):
            dma = pltpu.make_async_copy(
                w_hbm.at[:, pl.ds(qi * hh, hh)],
                w_ref.at[:, pl.ds(qi * hh, hh)],
                load_sems.at[i],
            )
            dma.start()
            w_dmas[qi] = dma
        chunk_order = [1, 3, 2, 0]
        chunk_dmas = {}
        for i, d in enumerate(chunk_order):
            j = lax.rem(my - d + N_DEV, N_DEV)
            dma = pltpu.make_async_copy(
                x_hbm.at[pl.ds(j * m_per, m_per), :],
                xs_ref.at[pl.ds(j * m_per, m_per), :],
                load_sems.at[4 + i],
            )
            dma.start()
            chunk_dmas[d] = dma

        barrier_sem = pltpu.get_barrier_semaphore()
        for nbr in (left, right):
            pl.semaphore_signal(
                barrier_sem,
                inc=1,
                device_id=(nbr,),
                device_id_type=pl.DeviceIdType.MESH,
            )
        pl.semaphore_wait(barrier_sem, 2)

        w_q = {}
        x_bf = {}

        def w_quarter(qi):
            if qi in w_dmas:
                w_dmas[qi].wait()
                del w_dmas[qi]
            if qi not in w_q:
                w_q[qi] = w_ref[:, qi * hh : (qi + 1) * hh].astype(jnp.bfloat16)
            return w_q[qi]

        def x_chunk(j, d):
            if d in chunk_dmas:
                chunk_dmas[d].wait()
                del chunk_dmas[d]
            if d not in x_bf:
                x_bf[d] = xs_ref[pl.ds(j * m_per, m_per), :].astype(jnp.bfloat16)
            return x_bf[d]

        def quarter_partial(ring, j, d):
            return jnp.dot(
                x_chunk(j, d),
                w_quarter(ring["wq"]),
                preferred_element_type=jnp.float32,
            ).astype(jnp.bfloat16)

        rdmas = {r["idx"]: [] for r in rings}
        for s in range(N_DEV - 1):
            jr = lax.rem(my - (s + 1) + N_DEV, N_DEV)
            jl = lax.rem(my + s + 1, N_DEV)
            for ring in rings:
                r = ring["idx"]
                j, d = (jr, s + 1) if ring["dir_right"] else (jl, 3 - s)
                part = quarter_partial(ring, j, d)
                if s == 0:
                    send0_ref[r] = part
                    src = send0_ref.at[r]
                else:
                    rdmas[r][s - 1].wait_recv()
                    recv_ref[r, s - 1] = recv_ref[r, s - 1] + part
                    src = recv_ref.at[r, s - 1]
                rdma = pltpu.make_async_remote_copy(
                    src_ref=src,
                    dst_ref=recv_ref.at[r, s],
                    send_sem=send_sems.at[r, s],
                    recv_sem=recv_sems.at[r, s],
                    device_id=(right if ring["dir_right"] else left,),
                    device_id_type=pl.DeviceIdType.MESH,
                )
                rdma.start()
                rdmas[r].append(rdma)

        amax = jnp.float32(0.0)
        for ring in rings:
            r = ring["idx"]
            part = quarter_partial(ring, my, 0)
            rdmas[r][-1].wait_recv()
            yb = recv_ref[r, N_DEV - 2] + part
            y = jnp.maximum(yb, jnp.zeros_like(yb))
            lo = ring["lo"]
            out_ref[:, lo : lo + hh] = y
            amax = jnp.maximum(amax, jnp.max(y.astype(jnp.float32)))

        amax_ref[0] = jnp.full((8, 128), amax, jnp.float32)
        amax_rdmas = []
        for d in range(1, N_DEV):
            peer = lax.rem(my + d, N_DEV)
            rr = pltpu.make_async_remote_copy(
                src_ref=amax_ref.at[0],
                dst_ref=amax_ref.at[d],
                send_sem=amax_send_sems.at[d - 1],
                recv_sem=amax_recv_sems.at[d - 1],
                device_id=(peer,),
                device_id_type=pl.DeviceIdType.MESH,
            )
            rr.start()
            amax_rdmas.append(rr)
        v = out_ref[...].astype(jnp.float32)
        for rr in amax_rdmas:
            rr.wait_recv()

        gmax = jnp.max(amax_ref[...])
        scale = gmax / 127.0
        inv = 127.0 / gmax
        q = jnp.round(jnp.clip(v * inv, 0.0, 127.0))
        out_ref[...] = (q * scale).astype(jnp.bfloat16)

        for r in rdmas:
            for rd in rdmas[r]:
                rd.wait_send()
        for rr in amax_rdmas:
            rr.wait_send()

    return pl.pallas_call(
        body,
        out_shape=jax.ShapeDtypeStruct((m_per, n), jnp.bfloat16),
        in_specs=[
            pl.BlockSpec(memory_space=pltpu.MemorySpace.HBM),
            pl.BlockSpec(memory_space=pltpu.MemorySpace.HBM),
        ],
        out_specs=pl.BlockSpec(memory_space=pltpu.VMEM),
        scratch_shapes=[
            pltpu.VMEM((m_global, k), jnp.float32),
            pltpu.VMEM((k, n), jnp.float32),
            pltpu.VMEM((4, m_per, hh), jnp.bfloat16),
            pltpu.VMEM((4, N_DEV - 1, m_per, hh), jnp.bfloat16),
            pltpu.VMEM((N_DEV, 8, 128), jnp.float32),
            pltpu.SemaphoreType.DMA((8,)),
            pltpu.SemaphoreType.DMA((4, N_DEV - 1)),
            pltpu.SemaphoreType.DMA((4, N_DEV - 1)),
            pltpu.SemaphoreType.DMA((N_DEV - 1,)),
            pltpu.SemaphoreType.DMA((N_DEV - 1,)),
        ],
        compiler_params=pltpu.CompilerParams(
            vmem_limit_bytes=128 * 1024 * 1024,
            collective_id=0,
        ),
    )(x, w_mat)
